# baseline (speedup 1.0000x reference)
import functools

import jax
import jax.numpy as jnp
from jax import lax
from jax.experimental import pallas as pl
from jax.experimental.pallas import tpu as pltpu
from jax.experimental.pallas import tpu_sc as plsc

_INFO = plsc.get_sparse_core_info()
_NC, _NS = _INFO.num_cores, _INFO.num_subcores
_NW = _NC * _NS
_L = 16

_CHUNK = 128


@functools.lru_cache(maxsize=None)
def _make_gather(B, V2, D2):
    b_per_w = B // _NW
    n_chunks = b_per_w // _CHUNK
    n_vec = b_per_w // _L
    o_per_w = b_per_w // 2

    mesh = plsc.VectorSubcoreMesh(core_axis_name="c", subcore_axis_name="s")

    @functools.partial(
        pl.kernel,
        out_type=jax.ShapeDtypeStruct((B // 2, D2), jnp.float32),
        mesh=mesh,
        scratch_types=[
            pltpu.VMEM((b_per_w,), jnp.int32),
            pltpu.VMEM((b_per_w,), jnp.int32),
            pltpu.VMEM((b_per_w,), jnp.int32),
            pltpu.VMEM((b_per_w, D2), jnp.float32),
            pltpu.VMEM((o_per_w, D2), jnp.float32),
            pltpu.SemaphoreType.DMA,
        ],
        compiler_params=pltpu.CompilerParams(
            use_tc_tiling_on_sc=True, needs_layout_passes=False),
    )
    def gather_kernel(idx_hbm, table_hbm, out_hbm, idx_v, blk_v, off_v,
                      rows_v, out_v, sem):
        wid = lax.axis_index("s") * _NC + lax.axis_index("c")
        pltpu.sync_copy(idx_hbm.at[pl.ds(wid * b_per_w, b_per_w)], idx_v)

        iota = lax.iota(jnp.int32, _L)

        @pl.loop(0, n_vec)
        def _prep(k):
            v = idx_v[pl.ds(k * _L, _L)]
            blk_v[pl.ds(k * _L, _L)] = lax.shift_right_logical(v, 1)
            off_v[pl.ds(k * _L, _L)] = lax.shift_left(
                lax.bitwise_and(v, 1), 6)

        copies = []
        for j in range(n_chunks):
            copies.append(
                pltpu.async_copy(
                    table_hbm.at[blk_v.at[pl.ds(j * _CHUNK, _CHUNK)]],
                    rows_v.at[pl.ds(j * _CHUNK, _CHUNK)],
                    sem,
                )
            )
        for c in copies:
            c.wait()

        half_iota = lax.shift_right_logical(iota, 1)
        par64 = lax.shift_left(lax.bitwise_and(iota, 1), 6)

        @pl.loop(0, n_vec)
        def _select(jb):
            src_row = jb * _L + iota
            dst_row = jb * (_L // 2) + half_iota
            offv = off_v[pl.ds(jb * _L, _L)]
            for c in range(D2 // 2):
                vals = plsc.load_gather(rows_v, [src_row, offv + c])
                plsc.store_scatter(out_v, [dst_row, par64 + c], vals)

        pltpu.sync_copy(out_v, out_hbm.at[pl.ds(wid * o_per_w, o_per_w)])

    return gather_kernel


def kernel(inst, embedding):
    B, = inst.shape
    V, D = embedding.shape
    table2 = embedding.reshape(V // 2, D * 2)
    out2 = _make_gather(B, V // 2, D * 2)(inst.astype(jnp.int32), table2)
    return out2.reshape(B, D)


# speedup vs baseline: 1.0841x; 1.0841x over previous
import functools

import jax
import jax.numpy as jnp
from jax import lax
from jax.experimental import pallas as pl
from jax.experimental.pallas import tpu as pltpu
from jax.experimental.pallas import tpu_sc as plsc

_INFO = plsc.get_sparse_core_info()
_NC, _NS = _INFO.num_cores, _INFO.num_subcores
_NW = _NC * _NS
_L = 16

_BLK = 128
_HCAP = 272
_SENT = _HCAP - 1
_RING = 4


def _full(x):
    return jnp.full((_L,), x, jnp.int32)


@functools.lru_cache(maxsize=None)
def _make_sweep(B, V, D):
    nblk_total = -(-V // _BLK)
    base_q, extra = divmod(nblk_total, _NW)
    first_extra = _NW - extra
    n_inter = B + _L

    mesh = plsc.VectorSubcoreMesh(core_axis_name="c", subcore_axis_name="s")

    @functools.partial(
        pl.kernel,
        out_type=jax.ShapeDtypeStruct((n_inter, 2 * D), jnp.float32),
        mesh=mesh,
        scratch_types=[
            pltpu.VMEM((B,), jnp.int32),
            pltpu.VMEM((B + _L,), jnp.int32),
            pltpu.VMEM((B + _L,), jnp.int32),
            pltpu.VMEM((_HCAP,), jnp.int32),
            pltpu.VMEM((_HCAP,), jnp.int32),
            pltpu.VMEM((_HCAP,), jnp.int32),
            pltpu.VMEM((2, D, _BLK), jnp.float32),
            pltpu.VMEM((_RING, _L, 2 * D), jnp.float32),
            pltpu.VMEM((_RING, _L), jnp.int32),
            pltpu.SemaphoreType.DMA,
            pltpu.SemaphoreType.DMA,
        ],
        compiler_params=pltpu.CompilerParams(
            use_tc_tiling_on_sc=True, needs_layout_passes=False),
    )
    def sweep_kernel(idx_hbm, tableT_hbm, tail_hbm, inter_hbm,
                     idx_v, ownv, ownp, hist, basep, fill,
                     chunk, rowb, posb, sem_in, sem_out):
        wid = lax.axis_index("s") * _NC + lax.axis_index("c")
        iota = lax.iota(jnp.int32, _L)
        onehot0 = (iota == 0).astype(jnp.int32)
        zeros16 = jnp.zeros((_L,), jnp.int32)
        ones16 = jnp.ones((_L,), jnp.int32)

        base_blk = base_q * wid + jnp.maximum(wid - first_extra, 0)
        nblk = base_q + (wid >= first_extra).astype(jnp.int32)
        lo = base_blk * _BLK
        hi = (base_blk + nblk) * _BLK

        pltpu.sync_copy(idx_hbm, idx_v)

        @pl.loop(0, _HCAP // _L)
        def _zero(k):
            hist[pl.ds(k * _L, _L)] = zeros16
            fill[pl.ds(k * _L, _L)] = zeros16

        @pl.loop(0, B // _L, unroll=4)
        def _hist(g):
            v = idx_v[pl.ds(g * _L, _L)]
            member = jnp.logical_and(v >= lo, v < hi)
            rb = jnp.where(member, lax.shift_right_logical(v - lo, 7), _SENT)
            plsc.addupdate_scatter(hist, [rb],
                                   jnp.where(member, ones16, zeros16))

        def _pfx(k, carry):
            seg = hist[pl.ds(k * _L, _L)]
            c = plsc.cumsum(seg)
            basep[pl.ds(k * _L, _L)] = c - seg + carry
            return carry + jnp.sum(seg)
        lax.fori_loop(0, _HCAP // _L, _pfx, jnp.int32(0))

        @pl.loop(0, B // _L, unroll=2)
        def _place(g):
            v = idx_v[pl.ds(g * _L, _L)]
            pos = g * _L + iota
            member = jnp.logical_and(v >= lo, v < hi)
            rb = jnp.where(member, lax.shift_right_logical(v - lo, 7), _SENT)
            cnt1, _ = plsc.scan_count(rb, mask=member)
            f = plsc.load_gather(fill, [rb])
            bs = plsc.load_gather(basep, [rb])
            slot = jnp.minimum(bs + f + cnt1 - 1, B + _L - 1)
            plsc.store_scatter(ownv, [slot], v, mask=member)
            plsc.store_scatter(ownp, [slot], pos, mask=member)
            plsc.addupdate_scatter(fill, [rb],
                                   jnp.where(member, ones16, zeros16))

        def _fire_load(b, buf):
            gblk = base_blk + b

            @pl.when(gblk != nblk_total - 1)
            def _():
                off = pl.multiple_of(gblk * _BLK, _BLK)
                pltpu.async_copy(tableT_hbm.at[:, pl.ds(off, _BLK)],
                                 chunk.at[buf], sem_in)

            @pl.when(gblk == nblk_total - 1)
            def _():
                pltpu.async_copy(tail_hbm, chunk.at[buf], sem_in)

        _fire_load(jnp.int32(0), jnp.int32(0))

        def _drain_in(buf):
            pltpu.make_async_copy(tableT_hbm.at[:, pl.ds(0, _BLK)],
                                  chunk.at[buf], sem_in).wait()

        def _drain_out(r):
            pltpu.make_async_copy(inter_hbm.at[pl.ds(0, _L)],
                                  rowb.at[r], sem_out).wait()

        def _block_body(b, s):
            cur = lax.rem(b, 2)
            _drain_in(cur)

            @pl.when(b + 1 < nblk)
            def _():
                _fire_load(b + 1, lax.rem(b + 1, 2))

            cnt_b = jnp.sum(hist[pl.ds(b, _L)] * onehot0)
            base_b = jnp.sum(basep[pl.ds(b, _L)] * onehot0)
            ngrp = lax.div(cnt_b + _L - 1, _L)

            def _grp(t, s):
                r = lax.rem(s, _RING)

                @pl.when(s >= _RING)
                def _():
                    _drain_out(r)

                mv = ownv[pl.ds(base_b + t * _L, _L)]
                mp = ownp[pl.ds(base_b + t * _L, _L)]
                valid = iota < (cnt_b - t * _L)
                il = lax.bitwise_and(mv, _BLK - 1)
                rf = _full(r)
                cf = _full(cur)
                for j in range(D):
                    vals = plsc.load_gather(chunk, [cf, _full(j), il])
                    plsc.store_scatter(rowb, [rf, iota, _full(j)], vals)
                posm = jnp.where(valid, mp, B + iota)
                plsc.store_scatter(posb, [rf, iota], posm)
                pltpu.async_copy(rowb.at[r], inter_hbm.at[posb.at[r]],
                                 sem_out)
                return s + 1

            return lax.fori_loop(0, ngrp, _grp, s)

        s_final = lax.fori_loop(0, nblk, _block_body, jnp.int32(0))

        def _final_drain(k, _):
            _drain_out(lax.rem(s_final - 1 - k, _RING))
            return 0
        lax.fori_loop(0, jnp.minimum(s_final, _RING), _final_drain, 0)

    return sweep_kernel


def kernel(inst, embedding):
    B, = inst.shape
    V, D = embedding.shape
    nfull = (V // _BLK) * _BLK
    tailT = jnp.pad(embedding[nfull:].T, ((0, 0), (0, _BLK - (V - nfull))))
    inter = _make_sweep(B, V, D)(inst.astype(jnp.int32), embedding.T, tailT)
    return inter[:B, :D]


# speedup vs baseline: 2.0972x; 1.9346x over previous
import functools

import jax
import jax.numpy as jnp
from jax import lax
from jax.experimental import pallas as pl
from jax.experimental.pallas import tpu as pltpu
from jax.experimental.pallas import tpu_sc as plsc

_INFO = plsc.get_sparse_core_info()
_NC, _NS = _INFO.num_cores, _INFO.num_subcores
_NW = _NC * _NS
_L = 16

_CW = 512
_CSH = 9
_HCAP = 80
_SENT = _HCAP - 1
_RING = 4


def _full(x):
    return jnp.full((_L,), x, jnp.int32)


@functools.lru_cache(maxsize=None)
def _make_sweep(B, V, D):
    nch_total = -(-V // _CW)
    base_q, extra = divmod(nch_total, _NW)
    first_extra = _NW - extra
    n_inter = B + _L

    mesh = plsc.VectorSubcoreMesh(core_axis_name="c", subcore_axis_name="s")

    @functools.partial(
        pl.kernel,
        out_type=jax.ShapeDtypeStruct((n_inter, 2 * D), jnp.float32),
        mesh=mesh,
        scratch_types=[
            pltpu.VMEM((B,), jnp.int32),
            pltpu.VMEM((B + _L,), jnp.int32),
            pltpu.VMEM((B + _L,), jnp.int32),
            pltpu.VMEM((_HCAP,), jnp.int32),
            pltpu.VMEM((_HCAP,), jnp.int32),
            pltpu.VMEM((_HCAP,), jnp.int32),
            pltpu.VMEM((2, D, _CW), jnp.float32),
            pltpu.VMEM((_RING, _L, 2 * D), jnp.float32),
            pltpu.VMEM((_RING, _L), jnp.int32),
            pltpu.SemaphoreType.DMA,
            pltpu.SemaphoreType.DMA,
        ],
        compiler_params=pltpu.CompilerParams(
            use_tc_tiling_on_sc=True, needs_layout_passes=False,
            disable_bounds_checks=True),
    )
    def sweep_kernel(idx_hbm, tableT_hbm, tail_hbm, inter_hbm,
                     idx_v, ownv, ownp, hist, basep, fill,
                     chunk, rowb, posb, sem_in, sem_out):
        wid = lax.axis_index("s") * _NC + lax.axis_index("c")
        iota = lax.iota(jnp.int32, _L)
        onehot0 = (iota == 0).astype(jnp.int32)
        zeros16 = jnp.zeros((_L,), jnp.int32)
        ones16 = jnp.ones((_L,), jnp.int32)

        base_ch = base_q * wid + jnp.maximum(wid - first_extra, 0)
        nch = base_q + (wid >= first_extra).astype(jnp.int32)
        lo = base_ch * _CW
        hi = (base_ch + nch) * _CW

        pltpu.sync_copy(idx_hbm, idx_v)

        @pl.loop(0, _HCAP // _L)
        def _zero(k):
            hist[pl.ds(k * _L, _L)] = zeros16
            fill[pl.ds(k * _L, _L)] = zeros16

        @pl.loop(0, B // _L, unroll=8)
        def _hist(g):
            v = idx_v[pl.ds(g * _L, _L)]
            member = jnp.logical_and(v >= lo, v < hi)
            rb = jnp.where(member, lax.shift_right_logical(v - lo, _CSH),
                           _SENT)
            plsc.addupdate_scatter(hist, [rb],
                                   jnp.where(member, ones16, zeros16))

        def _pfx(k, carry):
            seg = hist[pl.ds(k * _L, _L)]
            c = plsc.cumsum(seg)
            basep[pl.ds(k * _L, _L)] = c - seg + carry
            return carry + jnp.sum(seg)
        lax.fori_loop(0, _HCAP // _L, _pfx, jnp.int32(0))

        @pl.loop(0, B // _L, unroll=4)
        def _place(g):
            v = idx_v[pl.ds(g * _L, _L)]
            pos = g * _L + iota
            member = jnp.logical_and(v >= lo, v < hi)
            rb = jnp.where(member, lax.shift_right_logical(v - lo, _CSH),
                           _SENT)
            cnt1, _ = plsc.scan_count(rb, mask=member)
            f = plsc.load_gather(fill, [rb])
            bs = plsc.load_gather(basep, [rb])
            slot = jnp.minimum(bs + f + cnt1 - 1, B + _L - 1)
            plsc.store_scatter(ownv, [slot], v, mask=member)
            plsc.store_scatter(ownp, [slot], pos, mask=member)
            plsc.addupdate_scatter(fill, [rb],
                                   jnp.where(member, ones16, zeros16))

        def _fire_load(b, buf):
            gch = base_ch + b

            @pl.when(gch != nch_total - 1)
            def _():
                off = pl.multiple_of(gch * _CW, _CW)
                pltpu.async_copy(tableT_hbm.at[:, pl.ds(off, _CW)],
                                 chunk.at[buf], sem_in)

            @pl.when(gch == nch_total - 1)
            def _():
                pltpu.async_copy(tail_hbm, chunk.at[buf], sem_in)

        _fire_load(jnp.int32(0), jnp.int32(0))

        def _drain_in(buf):
            pltpu.make_async_copy(tableT_hbm.at[:, pl.ds(0, _CW)],
                                  chunk.at[buf], sem_in).wait()

        def _drain_out(r):
            pltpu.make_async_copy(inter_hbm.at[pl.ds(0, _L)],
                                  rowb.at[r], sem_out).wait()

        def _chunk_body(b, s):
            cur = lax.rem(b, 2)
            _drain_in(cur)

            @pl.when(b + 1 < nch)
            def _():
                _fire_load(b + 1, lax.rem(b + 1, 2))

            cnt_b = jnp.sum(hist[pl.ds(b, _L)] * onehot0)
            base_b = jnp.sum(basep[pl.ds(b, _L)] * onehot0)
            ngrp = lax.div(cnt_b + _L - 1, _L)

            def _grp(t, s):
                r = lax.rem(s, _RING)

                @pl.when(s >= _RING)
                def _():
                    _drain_out(r)

                mv = ownv[pl.ds(base_b + t * _L, _L)]
                mp = ownp[pl.ds(base_b + t * _L, _L)]
                valid = iota < (cnt_b - t * _L)
                il = lax.bitwise_and(mv, _CW - 1)
                rf = _full(r)
                cf = _full(cur)
                for j in range(D):
                    vals = plsc.load_gather(chunk, [cf, _full(j), il])
                    plsc.store_scatter(rowb, [rf, iota, _full(j)], vals)
                posm = jnp.where(valid, mp, B + iota)
                plsc.store_scatter(posb, [rf, iota], posm)
                pltpu.async_copy(rowb.at[r], inter_hbm.at[posb.at[r]],
                                 sem_out)
                return s + 1

            return lax.fori_loop(0, ngrp, _grp, s)

        s_final = lax.fori_loop(0, nch, _chunk_body, jnp.int32(0))

        def _final_drain(k, _):
            _drain_out(lax.rem(s_final - 1 - k, _RING))
            return 0
        lax.fori_loop(0, jnp.minimum(s_final, _RING), _final_drain, 0)

    return sweep_kernel


def kernel(inst, embedding):
    B, = inst.shape
    V, D = embedding.shape
    nfull = (V // _CW) * _CW
    tailT = jnp.pad(embedding[nfull:].T, ((0, 0), (0, _CW - (V - nfull))))
    inter = _make_sweep(B, V, D)(inst.astype(jnp.int32), embedding.T, tailT)
    return inter[:B, :D]


# speedup vs baseline: 2.1603x; 1.0301x over previous
import functools

import jax
import jax.numpy as jnp
from jax import lax
from jax.experimental import pallas as pl
from jax.experimental.pallas import tpu as pltpu
from jax.experimental.pallas import tpu_sc as plsc

_INFO = plsc.get_sparse_core_info()
_NC, _NS = _INFO.num_cores, _INFO.num_subcores
_NW = _NC * _NS
_L = 16

_CW = 1024
_CSH = 10
_HCAP = 48
_SENT = _HCAP - 1
_RING = 4


def _full(x):
    return jnp.full((_L,), x, jnp.int32)


@functools.lru_cache(maxsize=None)
def _make_sweep(B, V, D):
    nch_total = -(-V // _CW)
    base_q, extra = divmod(nch_total, _NW)
    first_extra = _NW - extra
    n_inter = B + _L

    mesh = plsc.VectorSubcoreMesh(core_axis_name="c", subcore_axis_name="s")

    DH = D // 2

    @functools.partial(
        pl.kernel,
        out_type=(jax.ShapeDtypeStruct((n_inter, 2 * D), jnp.float32),
                  jax.ShapeDtypeStruct((n_inter, 2 * D), jnp.float32)),
        mesh=mesh,
        scratch_types=[
            pltpu.VMEM((B,), jnp.int32),
            pltpu.VMEM((B + _L,), jnp.int32),
            pltpu.VMEM((B + _L,), jnp.int32),
            pltpu.VMEM((_HCAP,), jnp.int32),
            pltpu.VMEM((_HCAP,), jnp.int32),
            pltpu.VMEM((_HCAP,), jnp.int32),
            pltpu.VMEM((2, D // 2, _CW), jnp.float32),
            pltpu.VMEM((_RING, _L, 2 * D), jnp.float32),
            pltpu.VMEM((_RING, _L), jnp.int32),
            pltpu.SemaphoreType.DMA,
            pltpu.SemaphoreType.DMA,
        ],
        compiler_params=pltpu.CompilerParams(
            use_tc_tiling_on_sc=True, needs_layout_passes=False,
            disable_bounds_checks=True),
    )
    def sweep_kernel(idx_hbm, tableT_hbm, tail_hbm, inter0_hbm, inter1_hbm,
                     idx_v, ownv, ownp, hist, basep, fill,
                     chunk, rowb, posb, sem_in, sem_out):
        wid = lax.axis_index("s") * _NC + lax.axis_index("c")
        iota = lax.iota(jnp.int32, _L)
        onehot0 = (iota == 0).astype(jnp.int32)
        zeros16 = jnp.zeros((_L,), jnp.int32)
        ones16 = jnp.ones((_L,), jnp.int32)

        base_ch = base_q * wid + jnp.maximum(wid - first_extra, 0)
        nch = base_q + (wid >= first_extra).astype(jnp.int32)
        lo = base_ch * _CW
        hi = (base_ch + nch) * _CW

        pltpu.sync_copy(idx_hbm, idx_v)

        @pl.loop(0, _HCAP // _L)
        def _zero(k):
            hist[pl.ds(k * _L, _L)] = zeros16
            fill[pl.ds(k * _L, _L)] = zeros16

        @pl.loop(0, B // _L, unroll=8)
        def _hist(g):
            v = idx_v[pl.ds(g * _L, _L)]
            member = jnp.logical_and(v >= lo, v < hi)
            rb = jnp.where(member, lax.shift_right_logical(v - lo, _CSH),
                           _SENT)
            plsc.addupdate_scatter(hist, [rb],
                                   jnp.where(member, ones16, zeros16))

        def _pfx(k, carry):
            seg = hist[pl.ds(k * _L, _L)]
            c = plsc.cumsum(seg)
            basep[pl.ds(k * _L, _L)] = c - seg + carry
            return carry + jnp.sum(seg)
        lax.fori_loop(0, _HCAP // _L, _pfx, jnp.int32(0))

        @pl.loop(0, B // _L, unroll=4)
        def _place(g):
            v = idx_v[pl.ds(g * _L, _L)]
            pos = g * _L + iota
            member = jnp.logical_and(v >= lo, v < hi)
            rb = jnp.where(member, lax.shift_right_logical(v - lo, _CSH),
                           _SENT)
            cnt1, _ = plsc.scan_count(rb, mask=member)
            f = plsc.load_gather(fill, [rb])
            bs = plsc.load_gather(basep, [rb])
            slot = jnp.minimum(bs + f + cnt1 - 1, B + _L - 1)
            plsc.store_scatter(ownv, [slot], v, mask=member)
            plsc.store_scatter(ownp, [slot], pos, mask=member)
            plsc.addupdate_scatter(fill, [rb],
                                   jnp.where(member, ones16, zeros16))

        def _drain_out(r):
            pltpu.make_async_copy(inter0_hbm.at[pl.ds(0, _L)],
                                  rowb.at[r], sem_out).wait()

        s = jnp.int32(0)
        for h, inter_hbm in ((0, inter0_hbm), (1, inter1_hbm)):
            def _fire_load(b, buf, h=h):
                gch = base_ch + b

                @pl.when(gch != nch_total - 1)
                def _():
                    off = pl.multiple_of(gch * _CW, _CW)
                    pltpu.async_copy(
                        tableT_hbm.at[pl.ds(h * DH, DH), pl.ds(off, _CW)],
                        chunk.at[buf], sem_in)

                @pl.when(gch == nch_total - 1)
                def _():
                    pltpu.async_copy(tail_hbm.at[pl.ds(h * DH, DH), :],
                                     chunk.at[buf], sem_in)

            _fire_load(jnp.int32(0), jnp.int32(0))

            def _drain_in(buf):
                pltpu.make_async_copy(
                    tableT_hbm.at[pl.ds(0, DH), pl.ds(0, _CW)],
                    chunk.at[buf], sem_in).wait()

            def _chunk_body(b, s, _fire_load=_fire_load,
                            _drain_in=_drain_in, inter_hbm=inter_hbm):
                cur = lax.rem(b, 2)
                _drain_in(cur)

                @pl.when(b + 1 < nch)
                def _():
                    _fire_load(b + 1, lax.rem(b + 1, 2))

                cnt_b = jnp.sum(hist[pl.ds(b, _L)] * onehot0)
                base_b = jnp.sum(basep[pl.ds(b, _L)] * onehot0)
                ngrp = lax.div(cnt_b + _L - 1, _L)

                def _grp(t, s):
                    r = lax.rem(s, _RING)

                    @pl.when(s >= _RING)
                    def _():
                        _drain_out(r)

                    mv = ownv[pl.ds(base_b + t * _L, _L)]
                    mp = ownp[pl.ds(base_b + t * _L, _L)]
                    valid = iota < (cnt_b - t * _L)
                    il = lax.bitwise_and(mv, _CW - 1)
                    rf = _full(r)
                    cf = _full(cur)
                    for j in range(DH):
                        vals = plsc.load_gather(chunk, [cf, _full(j), il])
                        plsc.store_scatter(rowb, [rf, iota, _full(j)], vals)
                    posm = jnp.where(valid, mp, B + iota)
                    plsc.store_scatter(posb, [rf, iota], posm)
                    pltpu.async_copy(rowb.at[r], inter_hbm.at[posb.at[r]],
                                     sem_out)
                    return s + 1

                return lax.fori_loop(0, ngrp, _grp, s)

            s = lax.fori_loop(0, nch, _chunk_body, s)

        def _final_drain(k, _):
            _drain_out(lax.rem(s - 1 - k, _RING))
            return 0
        lax.fori_loop(0, jnp.minimum(s, _RING), _final_drain, 0)

    return sweep_kernel


def kernel(inst, embedding):
    B, = inst.shape
    V, D = embedding.shape
    nfull = (V // _CW) * _CW
    tailT = jnp.pad(embedding[nfull:].T, ((0, 0), (0, _CW - (V - nfull))))
    inter0, inter1 = _make_sweep(B, V, D)(
        inst.astype(jnp.int32), embedding.T, tailT)
    return jnp.concatenate([inter0[:B, :D // 2], inter1[:B, :D // 2]],
                           axis=1)
